# baseline (speedup 1.0000x reference)
import jax
import jax.numpy as jnp
from jax import lax
from jax.experimental import pallas as pl
from jax.experimental.pallas import tpu as pltpu
from jax.experimental.pallas import tpu_sc as plsc

_LANES = 16
_NUM_WORKERS = 32
_BATCH = 16384
_B_PER_W = _BATCH // _NUM_WORKERS
_N_TABLE = 1000


def _gather_body(t_hbm, table_hbm, out_hbm, idx_v, table_v, out_v):
    wid = lax.axis_index("s") * 2 + lax.axis_index("c")
    base = wid * _B_PER_W
    pltpu.sync_copy(table_hbm, table_v)
    pltpu.sync_copy(t_hbm.at[pl.ds(base, _B_PER_W)], idx_v)
    for i in range(_B_PER_W // _LANES):
        idx = idx_v[pl.ds(i * _LANES, _LANES)]
        out_v[pl.ds(i * _LANES, _LANES)] = plsc.load_gather(table_v, [idx])
    pltpu.sync_copy(out_v, out_hbm.at[pl.ds(base, _B_PER_W)])


@jax.jit
def kernel(t, alpha_cumprod):
    t = t.astype(jnp.int32)
    mesh = plsc.VectorSubcoreMesh(core_axis_name="c", subcore_axis_name="s")
    run = pl.kernel(
        _gather_body,
        mesh=mesh,
        out_type=jax.ShapeDtypeStruct((_BATCH,), jnp.float32),
        scratch_types=[
            pltpu.VMEM((_B_PER_W,), jnp.int32),
            pltpu.VMEM((_N_TABLE,), jnp.float32),
            pltpu.VMEM((_B_PER_W,), jnp.float32),
        ],
        compiler_params=pltpu.CompilerParams(needs_layout_passes=False),
    )
    return run(t, alpha_cumprod)


# speedup vs baseline: 1.0412x; 1.0412x over previous
import jax
import jax.numpy as jnp
from jax import lax
from jax.experimental import pallas as pl
from jax.experimental.pallas import tpu as pltpu
from jax.experimental.pallas import tpu_sc as plsc

_LANES = 16
_NUM_WORKERS = 32
_BATCH = 16384
_B_PER_W = _BATCH // _NUM_WORKERS
_N_TABLE = 1000


def _gather_body(t_hbm, table_hbm, out_hbm, idx_v, table_v, out_v, sem_t, sem_i):
    wid = lax.axis_index("s") * 2 + lax.axis_index("c")
    base = wid * _B_PER_W
    cp_t = pltpu.async_copy(table_hbm, table_v, sem_t)
    cp_i = pltpu.async_copy(t_hbm.at[pl.ds(base, _B_PER_W)], idx_v, sem_i)
    cp_t.wait()
    cp_i.wait()

    @plsc.parallel_loop(0, _B_PER_W, step=_LANES, unroll=4)
    def _(i):
        idx = idx_v[pl.ds(i, _LANES)]
        out_v[pl.ds(i, _LANES)] = plsc.load_gather(table_v, [idx])

    pltpu.sync_copy(out_v, out_hbm.at[pl.ds(base, _B_PER_W)])


@jax.jit
def kernel(t, alpha_cumprod):
    t = t.astype(jnp.int32)
    mesh = plsc.VectorSubcoreMesh(core_axis_name="c", subcore_axis_name="s")
    run = pl.kernel(
        _gather_body,
        mesh=mesh,
        out_type=jax.ShapeDtypeStruct((_BATCH,), jnp.float32),
        scratch_types=[
            pltpu.VMEM((_B_PER_W,), jnp.int32),
            pltpu.VMEM((_N_TABLE,), jnp.float32),
            pltpu.VMEM((_B_PER_W,), jnp.float32),
            pltpu.SemaphoreType.DMA,
            pltpu.SemaphoreType.DMA,
        ],
        compiler_params=pltpu.CompilerParams(needs_layout_passes=False),
    )
    return run(t, alpha_cumprod)


# speedup vs baseline: 1.0437x; 1.0024x over previous
import jax
import jax.numpy as jnp
from jax import lax
from jax.experimental import pallas as pl
from jax.experimental.pallas import tpu as pltpu
from jax.experimental.pallas import tpu_sc as plsc

_LANES = 16
_NUM_WORKERS = 32
_BATCH = 16384
_B_PER_W = _BATCH // _NUM_WORKERS
_N_TABLE = 1000


def _gather_body(t_hbm, table_hbm, out_hbm, idx_v, table_v, out_v, sem_t, sem_i, sem_o):
    wid = lax.axis_index("s") * 2 + lax.axis_index("c")
    base = wid * _B_PER_W
    half = _B_PER_W // 2
    cp_t = pltpu.async_copy(table_hbm, table_v, sem_t)
    cp_i = pltpu.async_copy(t_hbm.at[pl.ds(base, _B_PER_W)], idx_v, sem_i)
    cp_t.wait()
    cp_i.wait()

    @plsc.parallel_loop(0, half, step=_LANES, unroll=4)
    def _(i):
        idx = idx_v[pl.ds(i, _LANES)]
        out_v[pl.ds(i, _LANES)] = plsc.load_gather(table_v, [idx])

    cp_o1 = pltpu.async_copy(
        out_v.at[pl.ds(0, half)], out_hbm.at[pl.ds(base, half)], sem_o
    )

    @plsc.parallel_loop(half, _B_PER_W, step=_LANES, unroll=4)
    def _(i):
        idx = idx_v[pl.ds(i, _LANES)]
        out_v[pl.ds(i, _LANES)] = plsc.load_gather(table_v, [idx])

    cp_o2 = pltpu.async_copy(
        out_v.at[pl.ds(half, half)], out_hbm.at[pl.ds(base + half, half)], sem_o
    )
    cp_o1.wait()
    cp_o2.wait()


@jax.jit
def kernel(t, alpha_cumprod):
    t = t.astype(jnp.int32)
    mesh = plsc.VectorSubcoreMesh(core_axis_name="c", subcore_axis_name="s")
    run = pl.kernel(
        _gather_body,
        mesh=mesh,
        out_type=jax.ShapeDtypeStruct((_BATCH,), jnp.float32),
        scratch_types=[
            pltpu.VMEM((_B_PER_W,), jnp.int32),
            pltpu.VMEM((_N_TABLE,), jnp.float32),
            pltpu.VMEM((_B_PER_W,), jnp.float32),
            pltpu.SemaphoreType.DMA,
            pltpu.SemaphoreType.DMA,
            pltpu.SemaphoreType.DMA,
        ],
        compiler_params=pltpu.CompilerParams(needs_layout_passes=False),
    )
    return run(t, alpha_cumprod)
